# baseline (speedup 1.0000x reference)
import functools

import jax
import jax.numpy as jnp
from jax.experimental import pallas as pl

HIDDEN = 2048
EXPERTS = 64
K = 8
BT = 512


def _router_kernel(h_ref, w_ref, b_ref, logits_ref, idx_ref, wts_ref):
    h = h_ref[...]
    w = w_ref[...]
    logits = jax.lax.dot_general(
        h, w, (((1,), (1,)), ((), ())),
        preferred_element_type=jnp.float32,
    ) + b_ref[...][None, :]
    logits_ref[...] = logits

    iota = jax.lax.broadcasted_iota(jnp.int32, (BT, EXPERTS), 1)
    work = logits
    vals = []
    idxs = []
    for _ in range(K):
        m = jnp.max(work, axis=-1, keepdims=True)
        is_max = work == m
        idx = jnp.min(jnp.where(is_max, iota, EXPERTS), axis=-1, keepdims=True)
        vals.append(m)
        idxs.append(idx)
        work = jnp.where(iota == idx, -jnp.inf, work)
    top_v = jnp.concatenate(vals, axis=-1)
    top_i = jnp.concatenate(idxs, axis=-1)

    e = jnp.exp(top_v - top_v[:, :1])
    wts = e / jnp.sum(e, axis=-1, keepdims=True)

    idx_ref[...] = top_i
    wts_ref[...] = wts


@functools.partial(jax.jit, static_argnames=())
def kernel(hidden, W, b):
    n_tokens = hidden.shape[0]
    grid = (n_tokens // BT,)
    logits, idx, wts = pl.pallas_call(
        _router_kernel,
        grid=grid,
        in_specs=[
            pl.BlockSpec((BT, HIDDEN), lambda i: (i, 0)),
            pl.BlockSpec((EXPERTS, HIDDEN), lambda i: (0, 0)),
            pl.BlockSpec((EXPERTS,), lambda i: (0,)),
        ],
        out_specs=[
            pl.BlockSpec((BT, EXPERTS), lambda i: (i, 0)),
            pl.BlockSpec((BT, K), lambda i: (i, 0)),
            pl.BlockSpec((BT, K), lambda i: (i, 0)),
        ],
        out_shape=[
            jax.ShapeDtypeStruct((n_tokens, EXPERTS), jnp.float32),
            jax.ShapeDtypeStruct((n_tokens, K), jnp.int32),
            jax.ShapeDtypeStruct((n_tokens, K), jnp.float32),
        ],
    )(hidden, W, b)
    return idx, wts, logits


# speedup vs baseline: 1.5255x; 1.5255x over previous
import functools

import jax
import jax.numpy as jnp
from jax.experimental import pallas as pl

HIDDEN = 2048
EXPERTS = 64
K = 8
BT = 512


def _router_kernel(h_ref, w_ref, b_ref, logits_ref, idx_ref, wts_ref):
    h = h_ref[...]
    w = w_ref[...]
    b = b_ref[...]
    logits = jax.lax.dot_general(
        h, w, (((1,), (1,)), ((), ())),
        preferred_element_type=jnp.float32,
    ) + b[None, :]
    logits_ref[...] = logits

    lt = jax.lax.dot_general(
        w, h, (((1,), (1,)), ((), ())),
        preferred_element_type=jnp.float32,
    ) + b[:, None]

    iota = jax.lax.broadcasted_iota(jnp.int32, (EXPERTS, BT), 0)
    work = lt
    vals = []
    idxs = []
    for _ in range(K):
        m = jnp.max(work, axis=0, keepdims=True)
        is_max = work == m
        idx = jnp.min(jnp.where(is_max, iota, EXPERTS), axis=0, keepdims=True)
        vals.append(m)
        idxs.append(idx)
        work = jnp.where(iota == idx, -jnp.inf, work)
    top_v = jnp.concatenate(vals, axis=0)
    top_i = jnp.concatenate(idxs, axis=0)

    e = jnp.exp(top_v - top_v[:1])
    wts = e / jnp.sum(e, axis=0, keepdims=True)

    idx_ref[...] = top_i.T
    wts_ref[...] = wts.T


@functools.partial(jax.jit, static_argnames=())
def kernel(hidden, W, b):
    n_tokens = hidden.shape[0]
    grid = (n_tokens // BT,)
    logits, idx, wts = pl.pallas_call(
        _router_kernel,
        grid=grid,
        in_specs=[
            pl.BlockSpec((BT, HIDDEN), lambda i: (i, 0)),
            pl.BlockSpec((EXPERTS, HIDDEN), lambda i: (0, 0)),
            pl.BlockSpec((EXPERTS,), lambda i: (0,)),
        ],
        out_specs=[
            pl.BlockSpec((BT, EXPERTS), lambda i: (i, 0)),
            pl.BlockSpec((BT, K), lambda i: (i, 0)),
            pl.BlockSpec((BT, K), lambda i: (i, 0)),
        ],
        out_shape=[
            jax.ShapeDtypeStruct((n_tokens, EXPERTS), jnp.float32),
            jax.ShapeDtypeStruct((n_tokens, K), jnp.int32),
            jax.ShapeDtypeStruct((n_tokens, K), jnp.float32),
        ],
    )(hidden, W, b)
    return idx, wts, logits


# speedup vs baseline: 1.7578x; 1.1523x over previous
import functools

import jax
import jax.numpy as jnp
from jax.experimental import pallas as pl

HIDDEN = 2048
EXPERTS = 64
K = 8
BT = 1024


def _router_kernel(h_ref, w_ref, b_ref, logits_ref, idx_ref, wts_ref):
    h = h_ref[...]
    w = w_ref[...]
    b = b_ref[...]
    logits = jax.lax.dot_general(
        h, w, (((1,), (1,)), ((), ())),
        preferred_element_type=jnp.float32,
    ) + b[None, :]
    logits_ref[...] = logits

    lt = jax.lax.dot_general(
        w, h, (((1,), (1,)), ((), ())),
        preferred_element_type=jnp.float32,
    ) + b[:, None]

    iota = jax.lax.broadcasted_iota(jnp.int32, (EXPERTS, BT), 0)
    work = lt
    vals = []
    idxs = []
    for _ in range(K):
        m = jnp.max(work, axis=0, keepdims=True)
        is_max = work == m
        idx = jnp.min(jnp.where(is_max, iota, EXPERTS), axis=0, keepdims=True)
        vals.append(m)
        idxs.append(idx)
        work = jnp.where(iota == idx, -jnp.inf, work)
    top_v = jnp.concatenate(vals, axis=0)
    top_i = jnp.concatenate(idxs, axis=0)

    e = jnp.exp(top_v - top_v[:1])
    wts = e / jnp.sum(e, axis=0, keepdims=True)

    idx_ref[...] = top_i.T
    wts_ref[...] = wts.T


@functools.partial(jax.jit, static_argnames=())
def kernel(hidden, W, b):
    n_tokens = hidden.shape[0]
    grid = (n_tokens // BT,)
    logits, idx, wts = pl.pallas_call(
        _router_kernel,
        grid=grid,
        in_specs=[
            pl.BlockSpec((BT, HIDDEN), lambda i: (i, 0)),
            pl.BlockSpec((EXPERTS, HIDDEN), lambda i: (0, 0)),
            pl.BlockSpec((EXPERTS,), lambda i: (0,)),
        ],
        out_specs=[
            pl.BlockSpec((BT, EXPERTS), lambda i: (i, 0)),
            pl.BlockSpec((BT, K), lambda i: (i, 0)),
            pl.BlockSpec((BT, K), lambda i: (i, 0)),
        ],
        out_shape=[
            jax.ShapeDtypeStruct((n_tokens, EXPERTS), jnp.float32),
            jax.ShapeDtypeStruct((n_tokens, K), jnp.int32),
            jax.ShapeDtypeStruct((n_tokens, K), jnp.float32),
        ],
    )(hidden, W, b)
    return idx, wts, logits


# speedup vs baseline: 1.8436x; 1.0488x over previous
import functools

import jax
import jax.numpy as jnp
from jax.experimental import pallas as pl

HIDDEN = 2048
EXPERTS = 64
K = 8
BT = 2048


def _router_kernel(h_ref, w_ref, b_ref, logits_ref, idx_ref, wts_ref):
    h = h_ref[...]
    w = w_ref[...]
    b = b_ref[...]
    logits = jax.lax.dot_general(
        h, w, (((1,), (1,)), ((), ())),
        preferred_element_type=jnp.float32,
    ) + b[None, :]
    logits_ref[...] = logits

    lt = jax.lax.dot_general(
        w, h, (((1,), (1,)), ((), ())),
        preferred_element_type=jnp.float32,
    ) + b[:, None]

    iota = jax.lax.broadcasted_iota(jnp.int32, (EXPERTS, BT), 0)
    work = lt
    vals = []
    idxs = []
    for _ in range(K):
        m = jnp.max(work, axis=0, keepdims=True)
        is_max = work == m
        idx = jnp.min(jnp.where(is_max, iota, EXPERTS), axis=0, keepdims=True)
        vals.append(m)
        idxs.append(idx)
        work = jnp.where(iota == idx, -jnp.inf, work)
    top_v = jnp.concatenate(vals, axis=0)
    top_i = jnp.concatenate(idxs, axis=0)

    e = jnp.exp(top_v - top_v[:1])
    wts = e / jnp.sum(e, axis=0, keepdims=True)

    idx_ref[...] = top_i.T
    wts_ref[...] = wts.T


@functools.partial(jax.jit, static_argnames=())
def kernel(hidden, W, b):
    n_tokens = hidden.shape[0]
    grid = (n_tokens // BT,)
    logits, idx, wts = pl.pallas_call(
        _router_kernel,
        grid=grid,
        in_specs=[
            pl.BlockSpec((BT, HIDDEN), lambda i: (i, 0)),
            pl.BlockSpec((EXPERTS, HIDDEN), lambda i: (0, 0)),
            pl.BlockSpec((EXPERTS,), lambda i: (0,)),
        ],
        out_specs=[
            pl.BlockSpec((BT, EXPERTS), lambda i: (i, 0)),
            pl.BlockSpec((BT, K), lambda i: (i, 0)),
            pl.BlockSpec((BT, K), lambda i: (i, 0)),
        ],
        out_shape=[
            jax.ShapeDtypeStruct((n_tokens, EXPERTS), jnp.float32),
            jax.ShapeDtypeStruct((n_tokens, K), jnp.int32),
            jax.ShapeDtypeStruct((n_tokens, K), jnp.float32),
        ],
    )(hidden, W, b)
    return idx, wts, logits
